# Initial kernel scaffold; baseline (speedup 1.0000x reference)
import jax
import jax.numpy as jnp
from jax.experimental import pallas as pl


def kernel(x, edge_index, edge_attr, batch, idx, params):
    raise NotImplementedError("write your pallas kernel here")


# speedup vs baseline: 2.1915x; 2.1915x over previous
import functools
import jax
import jax.numpy as jnp
from jax import lax
from jax.experimental import pallas as pl
from jax.experimental.pallas import tpu as pltpu
from jax.experimental.pallas import tpu_sc as plsc

NN = 10000
EE = 160000
NB = 100
NMAXC = 100
F = 16
C = 4
LATD = 64
COSD = 8
SETC_ = 32
HD = 64
HE = 32
SD = 3

NC = 2
NS = 16
NW = NC * NS
LANES = 16
CH = 128
NCH = EE // CH
TPW = (NCH + NW - 1) // NW
NNP = 10240
RPW = NNP // NS

def _sc_gather_body(h_hbm, src_hbm, dst_hbm, eaf_hbm, batch_hbm, zn_hbm,
                    zec_hbm, hsrc_hbm, degp_hbm, ecp_hbm,
                    sidx, didx, bsrcv, eav, rows, btab, dtab, ectab, sem):
    wid = lax.axis_index("s") * NC + lax.axis_index("c")
    pltpu.sync_copy(batch_hbm, btab)
    pltpu.sync_copy(zn_hbm, dtab)
    pltpu.sync_copy(zec_hbm, ectab)
    iota = lax.iota(jnp.int32, LANES)
    rep4 = iota // 4
    mod4 = iota - rep4 * 4
    ones = jnp.full((LANES,), 1.0, jnp.float32)

    def body(t, carry):
        ch = t * NW + wid

        @pl.when(ch < NCH)
        def _():
            e0 = ch * CH
            pltpu.sync_copy(src_hbm.at[pl.ds(e0, CH)], sidx)
            pltpu.sync_copy(dst_hbm.at[pl.ds(e0, CH)], didx)
            pltpu.sync_copy(eaf_hbm.at[pl.ds(e0 * C, CH * C)], eav)
            pltpu.async_copy(h_hbm.at[sidx], rows, sem).wait()
            pltpu.sync_copy(rows, hsrc_hbm.at[pl.ds(e0, CH)])
            for j in range(CH // LANES):
                d16 = didx[pl.ds(j * LANES, LANES)]
                plsc.addupdate_scatter(dtab, [d16], ones)
                s16 = sidx[pl.ds(j * LANES, LANES)]
                bsrcv[pl.ds(j * LANES, LANES)] = plsc.load_gather(btab, [s16])
            for g in range(CH // 4):
                ridx = g * 4 + rep4
                bg = plsc.load_gather(bsrcv, [ridx])
                vals = eav[pl.ds(g * LANES, LANES)]
                plsc.addupdate_scatter(ectab, [bg * C + mod4], vals)

        return carry

    lax.fori_loop(0, TPW, body, 0)
    pltpu.sync_copy(dtab, degp_hbm.at[wid])
    pltpu.sync_copy(ectab, ecp_hbm.at[wid])


def _sc_scatter_body(msg_hbm, dst_hbm, zn16_hbm, aggp_hbm,
                     didx, rowsv, shacc, sem):
    cid = lax.axis_index("c")
    sid = lax.axis_index("s")
    wid = sid * NC + cid
    rpw = RPW

    pltpu.sync_copy(zn16_hbm.at[pl.ds(sid * rpw, rpw)],
                    shacc.at[pl.ds(sid * rpw, rpw)])
    plsc.subcore_barrier()

    def body(t, carry):
        ch = t * NW + wid

        @pl.when(ch < NCH)
        def _():
            e0 = ch * CH
            pltpu.sync_copy(dst_hbm.at[pl.ds(e0, CH)], didx)
            pltpu.sync_copy(msg_hbm.at[pl.ds(e0, CH)], rowsv)
            pltpu.sync_copy(rowsv, shacc.at[didx], add=True)

        return carry

    lax.fori_loop(0, TPW, body, 0)
    plsc.subcore_barrier()
    pltpu.sync_copy(shacc.at[pl.ds(sid * rpw, rpw)],
                    aggp_hbm.at[pl.ds(cid * NNP + sid * rpw, rpw)])


@functools.cache
def _sc_kernels():
    mesh = plsc.VectorSubcoreMesh(core_axis_name="c", subcore_axis_name="s",
                                  num_cores=NC, num_subcores=NS)
    gather = pl.kernel(
        _sc_gather_body,
        out_type=(
            jax.ShapeDtypeStruct((EE, F), jnp.float32),
            jax.ShapeDtypeStruct((NW, NN), jnp.float32),
            jax.ShapeDtypeStruct((NW, NB * C), jnp.float32),
        ),
        mesh=mesh,
        compiler_params=pltpu.CompilerParams(use_tc_tiling_on_sc=False, needs_layout_passes=False),
        scratch_types=[
            pltpu.VMEM((CH,), jnp.int32),
            pltpu.VMEM((CH,), jnp.int32),
            pltpu.VMEM((CH,), jnp.int32),
            pltpu.VMEM((CH * C,), jnp.float32),
            pltpu.VMEM((CH, F), jnp.float32),
            pltpu.VMEM((NN,), jnp.int32),
            pltpu.VMEM((NN,), jnp.float32),
            pltpu.VMEM((NB * C,), jnp.float32),
            pltpu.SemaphoreType.DMA,
        ],
    )
    scatter = pl.kernel(
        _sc_scatter_body,
        out_type=jax.ShapeDtypeStruct((NC * NNP, F), jnp.float32),
        mesh=mesh,
        compiler_params=pltpu.CompilerParams(use_tc_tiling_on_sc=False, needs_layout_passes=False),
        scratch_types=[
            pltpu.VMEM((CH,), jnp.int32),
            pltpu.VMEM((CH, F), jnp.float32),
            pltpu.VMEM_SHARED((NNP, F), jnp.float32),
            pltpu.SemaphoreType.DMA,
        ],
    )
    return gather, scatter


def _relu(v):
    return jnp.maximum(v, 0.0)


def _dot(a, b):
    return jnp.dot(a, b, preferred_element_type=jnp.float32)


def _dg(a, b, ca, cb):
    return lax.dot_general(a, b, (((ca,), (cb,)), ((), ())),
                           preferred_element_type=jnp.float32)


def _encinit_body(x_ref, w0, b0, w1, b1, w2, b2, h_ref):
    h = _relu(_dot(x_ref[...], w0[...]) + b0[...])
    h = _relu(_dot(h, w1[...]) + b1[...])
    h_ref[...] = _dot(h, w2[...]) + b2[...]


def _msg_body(hs_ref, ea_ref, w2s_ref, bmat_ref, msg_ref):
    hs = hs_ref[...]
    acc = _dot(hs, bmat_ref[...])
    for c in range(C):
        acc = acc + _dot(hs * ea_ref[:, c][:, None], w2s_ref[c])
    msg_ref[...] = acc


def _enctail_body(x_ref, h_ref, aggp_ref, degp_ref, ecp_ref, batch_ref,
                  eps_ref, rootW, rootb, Wc, We4, Wa, Wm, Wx, Wml, Wxl,
                  b0r, W1r, b1r, Wfr, bfr,
                  mu_ref, lv_ref, lat_ref, ohT_s, cnt_s):
    x = x_ref[...]
    h = h_ref[...]
    ones_nw = jnp.full((NW, 1), 1.0, jnp.float32)
    deg = _dg(degp_ref[...], ones_nw, 0, 0)
    agg = (aggp_ref[0:NN, :] + aggp_ref[NNP:NNP + NN, :]) / jnp.maximum(deg, 1.0)
    new_x = _dot(h, rootW[...]) + rootb[...] + agg

    ec = jnp.sum(ecp_ref[...], axis=0) * (1.0 / NMAXC)

    batr = batch_ref[...]
    iota_bn = lax.broadcasted_iota(jnp.int32, (NB, NN), 0)
    ohT = jnp.where(iota_bn == batr, 1.0, 0.0)
    counts = jnp.sum(ohT, axis=1, keepdims=True)
    safe = jnp.maximum(counts, 1.0)
    addp = _dg(ohT, x, 1, 0)
    suml = _dg(ohT, new_x, 1, 0)
    meanp = addp / safe
    meanl = suml / safe

    ii = lax.broadcasted_iota(jnp.int32, (F, F), 0)
    jj = lax.broadcasted_iota(jnp.int32, (F, F), 1)
    eye = jnp.where(ii == jj, 1.0, 0.0)
    xT = _dg(eye, x, 1, 1)
    nxT = _dg(eye, new_x, 1, 1)

    ohT_s[...] = ohT
    cnt_s[...] = counts
    laneB = lax.broadcasted_iota(jnp.int32, (F, NB), 1)
    big = jnp.float32(-3.0e38)

    def mbody(b, accs):
        mp, ml = accs
        mask = ohT_s[pl.ds(b, 1), :] > 0.5
        cntb = cnt_s[pl.ds(b, 1), :]
        mx = jnp.max(jnp.where(mask, xT, big), axis=1, keepdims=True)
        mx = jnp.where(cntb > 0.5, mx, 0.0)
        mlx = jnp.max(jnp.where(mask, nxT, big), axis=1, keepdims=True)
        mlx = jnp.where(cntb > 0.5, mlx, 0.0)
        sel = laneB == b
        return (jnp.where(sel, mx, mp), jnp.where(sel, mlx, ml))

    zf = jnp.zeros((F, NB), jnp.float32)
    maxpT, maxlT = lax.fori_loop(0, NB, mbody, (zf, zf))

    h0 = (counts * (1.0 / NMAXC)) * Wc[...]
    h0 = h0 + _dot(ec, We4[...])
    h0 = h0 + _dot(addp * (1.0 / NMAXC), Wa[...])
    h0 = h0 + _dot(meanp, Wm[...])
    h0 = h0 + _dg(maxpT, Wx[...], 0, 0)
    h0 = h0 + _dot(meanl, Wml[...])
    h0 = h0 + _dg(maxlT, Wxl[...], 0, 0)
    h0 = _relu(h0 + b0r[...])
    h1 = _relu(_dot(h0, W1r[...]) + b1r[...])
    eo = _dot(h1, Wfr[...]) + bfr[...]
    mu = eo[:, :LATD]
    lv = eo[:, LATD:]
    mu_ref[...] = mu
    lv_ref[...] = lv
    lat_ref[...] = mu + eps_ref[...] * jnp.exp(0.5 * lv)


def _dec_body(lat_ref, pts_ref, Sj_ref,
              W0, b0, W0b, b0b, Wh, bh, Wfin, bfin,
              Wd0, bd0, Wd1, bd1,
              Wl0, bl0, Wl1, bl1, Wlf, blf,
              WeA, WeB, be0, We1, be1, Wef, befT,
              WaP, WaS, ba0, Wa1, ba1, Waf, baf,
              la_ref, ep_ref, et_ref):
    lat = lat_ref[0]
    y = lat[:, COSD:]
    t = _relu(_dot(pts_ref[...], W0[...]) + b0[...] + _dot(y, W0b[...]) + b0b[...])
    t = _relu(_dot(t, Wh[...]) + bh[...])
    z = _relu(_dot(t, Wfin[...]) + bfin[...])
    z = z + _relu(_dot(z, Wd0[...]) + bd0[...])
    z = z + _relu(_dot(z, Wd1[...]) + bd1[...])
    t2 = _relu(_dot(z, Wl0[...]) + bl0[...])
    t2 = _relu(_dot(t2, Wl1[...]) + bl1[...])
    pos = _dot(t2, Wlf[...]) + blf[...]

    h1 = _dot(pos, WeA[...]) + be0[...]
    h2 = _dot(pos, WeB[...])
    hh = _relu((h1[:, None, :] + h2[None, :, :]).reshape(NMAXC * NMAXC, HE))
    hh = _relu(_dot(hh, We1[...]) + be1[...])
    lgT = _dg(Wef[...], hh, 0, 1) + befT[...]
    l4 = lgT[:C, :]
    l5 = lgT[C:, :]
    m4 = jnp.max(l4, axis=0, keepdims=True)
    s4 = jnp.sum(jnp.exp(l4 - m4), axis=0, keepdims=True)
    etT = l4 - m4 - jnp.log(s4)
    m5 = jnp.maximum(m4, l5)
    s5 = jnp.sum(jnp.exp(l4 - m5), axis=0, keepdims=True) + jnp.exp(l5 - m5)
    epT = 1.0 - jnp.exp(l5 - m5) / s5

    sump2 = _dg(etT, Sj_ref[...], 1, 0) * (1.0 / 9.0)
    a0 = _relu(_dot(pos, WaP[...]) + _dg(sump2, WaS[...], 0, 0) + ba0[...])
    a1 = _relu(_dot(a0, Wa1[...]) + ba1[...])
    la = _dot(a1, Waf[...]) + baf[...]
    mla = jnp.max(la, axis=1, keepdims=True)
    la = la - mla - jnp.log(jnp.sum(jnp.exp(la - mla), axis=1, keepdims=True))

    la_ref[0] = la
    ep_ref[0] = epT
    et_ref[0] = etT


def _row(b):
    return b.reshape(1, -1)


def kernel(x, edge_index, edge_attr, batch, idx, params):
    f32 = jnp.float32
    src = edge_index[0].astype(jnp.int32)
    dst = edge_index[1].astype(jnp.int32)
    eaf = edge_attr.reshape(-1)
    batch = batch.astype(jnp.int32)
    zn = jnp.zeros((NN,), f32)
    zec = jnp.zeros((NB * C,), f32)
    zn16 = jnp.zeros((NNP, F), f32)
    eps = jax.random.normal(jax.random.key(1), (NB, LATD), dtype=f32)

    p = params
    ei = p["enc_init"]
    ef = p["enc_final"]
    W2s = p["nn_W"].reshape(C, F, F)
    Bmat = p["nn_b"].reshape(F, F)
    Wefin = ef["lin0"][0]

    h = pl.pallas_call(
        _encinit_body,
        out_shape=jax.ShapeDtypeStruct((NN, F), f32),
    )(x, ei["lin0"][0], _row(ei["lin0"][1]),
      ei["hidden"][0][0], _row(ei["hidden"][0][1]),
      ei["fin"][0], _row(ei["fin"][1]))

    sc_gather, sc_scatter = _sc_kernels()
    hsrc, degp, ecp = sc_gather(h, src, dst, eaf, batch, zn, zec)

    BE = 4000
    msg = pl.pallas_call(
        _msg_body,
        grid=(EE // BE,),
        in_specs=[
            pl.BlockSpec((BE, F), lambda e: (e, 0)),
            pl.BlockSpec((BE, C), lambda e: (e, 0)),
            pl.BlockSpec((C, F, F), lambda e: (0, 0, 0)),
            pl.BlockSpec((F, F), lambda e: (0, 0)),
        ],
        out_specs=pl.BlockSpec((BE, F), lambda e: (e, 0)),
        out_shape=jax.ShapeDtypeStruct((EE, F), f32),
    )(hsrc, edge_attr, W2s, Bmat)

    aggp = sc_scatter(msg, dst, zn16)

    mu, log_var, latent = pl.pallas_call(
        _enctail_body,
        out_shape=(
            jax.ShapeDtypeStruct((NB, LATD), f32),
            jax.ShapeDtypeStruct((NB, LATD), f32),
            jax.ShapeDtypeStruct((NB, LATD), f32),
        ),
        scratch_shapes=[
            pltpu.VMEM((NB, NN), f32),
            pltpu.VMEM((NB, 1), f32),
        ],
    )(x, h, aggp, degp, ecp.reshape(NW, NB, C), batch.reshape(1, NN), eps,
      p["root_W"], _row(p["root_b"]),
      Wefin[0:1], Wefin[1:1 + C], Wefin[5:21], Wefin[21:37], Wefin[37:53],
      Wefin[53:69], Wefin[69:85],
      _row(ef["lin0"][1]), ef["hidden"][0][0], _row(ef["hidden"][0][1]),
      ef["fin"][0], _row(ef["fin"][1]))

    di = p["dec_init"]
    ls = p["last_set"]
    em = p["edge_mlp"]
    am = p["atom_mlp"]
    Sj = (jnp.arange(NMAXC * NMAXC, dtype=jnp.int32)[:, None] // NMAXC
          == jnp.arange(NMAXC, dtype=jnp.int32)[None, :]).astype(f32)
    full = lambda *s: pl.BlockSpec(s, lambda g: tuple(0 for _ in s))

    la_out, ep_out, etT_out = pl.pallas_call(
        _dec_body,
        grid=(NB,),
        in_specs=[
            pl.BlockSpec((1, 1, LATD), lambda g: (g, 0, 0)),
            full(NMAXC, SETC_),
            full(NMAXC * NMAXC, NMAXC),
            full(SETC_, HD), full(1, HD),
            full(LATD - COSD, HD), full(1, HD),
            full(HD, HD), full(1, HD),
            full(HD, HD), full(1, HD),
            full(HD, HD), full(1, HD),
            full(HD, HD), full(1, HD),
            full(HD, HD), full(1, HD),
            full(HD, HD), full(1, HD),
            full(HD, SD), full(1, SD),
            full(SD, HE), full(SD, HE), full(1, HE),
            full(HE, HE), full(1, HE),
            full(HE, C + 1), full(C + 1, 1),
            full(SD, HD), full(C, HD), full(1, HD),
            full(HD, HD), full(1, HD),
            full(HD, F), full(1, F),
        ],
        out_specs=(
            pl.BlockSpec((1, NMAXC, F), lambda g: (g, 0, 0)),
            pl.BlockSpec((1, 1, NMAXC * NMAXC), lambda g: (g, 0, 0)),
            pl.BlockSpec((1, C, NMAXC * NMAXC), lambda g: (g, 0, 0)),
        ),
        out_shape=(
            jax.ShapeDtypeStruct((NB, NMAXC, F), f32),
            jax.ShapeDtypeStruct((NB, 1, NMAXC * NMAXC), f32),
            jax.ShapeDtypeStruct((NB, C, NMAXC * NMAXC), f32),
        ),
    )(latent.reshape(NB, 1, LATD), p["points"], Sj,
      di["lin0"][0], _row(di["lin0"][1]),
      di["lin0b"][0], _row(di["lin0b"][1]),
      di["hidden"][0][0], _row(di["hidden"][0][1]),
      di["fin"][0], _row(di["fin"][1]),
      p["dec_lin0"][0], _row(p["dec_lin0"][1]),
      p["dec_lin1"][0], _row(p["dec_lin1"][1]),
      ls["lin0"][0], _row(ls["lin0"][1]),
      ls["hidden"][0][0], _row(ls["hidden"][0][1]),
      ls["fin"][0], _row(ls["fin"][1]),
      em["lin0"][0][:SD], em["lin0"][0][SD:], _row(em["lin0"][1]),
      em["hidden"][0][0], _row(em["hidden"][0][1]),
      em["fin"][0], em["fin"][1].reshape(C + 1, 1),
      am["lin0"][0][:SD], am["lin0"][0][SD:], _row(am["lin0"][1]),
      am["hidden"][0][0], _row(am["hidden"][0][1]),
      am["fin"][0], _row(am["fin"][1]))

    edge_probs = ep_out.reshape(NB, NMAXC, NMAXC)
    edge_types = jnp.transpose(etT_out, (0, 2, 1)).reshape(NB, NMAXC, NMAXC, C)
    return (la_out, edge_probs, edge_types, mu, log_var)


# speedup vs baseline: 2.5257x; 1.1525x over previous
import functools
import jax
import jax.numpy as jnp
from jax import lax
from jax.experimental import pallas as pl
from jax.experimental.pallas import tpu as pltpu
from jax.experimental.pallas import tpu_sc as plsc

NN = 10000
EE = 160000
NB = 100
NMAXC = 100
F = 16
C = 4
LATD = 64
COSD = 8
SETC_ = 32
HD = 64
HE = 32
SD = 3

NC = 2
NS = 16
NW = NC * NS
LANES = 16
CH = 128
NCH = EE // CH
TPW = (NCH + NW - 1) // NW
NNP = 10240
RPW = NNP // NS
GDEC = 4

def _sc_gather_body(h_hbm, src_hbm, dst_hbm, eaf_hbm, batch_hbm, zn_hbm,
                    zec_hbm, hsrc_hbm, degp_hbm, ecp_hbm,
                    sidx, didx, bsrcv, eav, rows, btab, dtab, ectab, sem):
    wid = lax.axis_index("s") * NC + lax.axis_index("c")
    pltpu.sync_copy(batch_hbm, btab)
    pltpu.sync_copy(zn_hbm, dtab)
    pltpu.sync_copy(zec_hbm, ectab)
    iota = lax.iota(jnp.int32, LANES)
    rep4 = iota // 4
    mod4 = iota - rep4 * 4
    ones = jnp.full((LANES,), 1.0, jnp.float32)

    def body(t, carry):
        ch = t * NW + wid

        @pl.when(ch < NCH)
        def _():
            e0 = ch * CH
            pltpu.sync_copy(src_hbm.at[pl.ds(e0, CH)], sidx)
            pltpu.sync_copy(dst_hbm.at[pl.ds(e0, CH)], didx)
            pltpu.sync_copy(eaf_hbm.at[pl.ds(e0 * C, CH * C)], eav)
            pltpu.async_copy(h_hbm.at[sidx], rows, sem).wait()
            pltpu.sync_copy(rows, hsrc_hbm.at[pl.ds(e0, CH)])
            for j in range(CH // LANES):
                d16 = didx[pl.ds(j * LANES, LANES)]
                plsc.addupdate_scatter(dtab, [d16], ones)
                s16 = sidx[pl.ds(j * LANES, LANES)]
                bsrcv[pl.ds(j * LANES, LANES)] = plsc.load_gather(btab, [s16])
            for g in range(CH // 4):
                ridx = g * 4 + rep4
                bg = plsc.load_gather(bsrcv, [ridx])
                vals = eav[pl.ds(g * LANES, LANES)]
                plsc.addupdate_scatter(ectab, [bg * C + mod4], vals)

        return carry

    lax.fori_loop(0, TPW, body, 0)
    pltpu.sync_copy(dtab, degp_hbm.at[wid])
    pltpu.sync_copy(ectab, ecp_hbm.at[wid])


def _sc_scatter_body(msg_hbm, dst_hbm, zn16_hbm, aggp_hbm,
                     didx, rowsv, shacc, sem):
    cid = lax.axis_index("c")
    sid = lax.axis_index("s")
    wid = sid * NC + cid
    rpw = RPW

    pltpu.sync_copy(zn16_hbm.at[pl.ds(sid * rpw, rpw)],
                    shacc.at[pl.ds(sid * rpw, rpw)])
    plsc.subcore_barrier()

    def body(t, carry):
        ch = t * NW + wid

        @pl.when(ch < NCH)
        def _():
            e0 = ch * CH
            pltpu.sync_copy(dst_hbm.at[pl.ds(e0, CH)], didx)
            pltpu.sync_copy(msg_hbm.at[pl.ds(e0, CH)], rowsv)
            pltpu.sync_copy(rowsv, shacc.at[didx], add=True)

        return carry

    lax.fori_loop(0, TPW, body, 0)
    plsc.subcore_barrier()
    pltpu.sync_copy(shacc.at[pl.ds(sid * rpw, rpw)],
                    aggp_hbm.at[pl.ds(cid * NNP + sid * rpw, rpw)])


@functools.cache
def _sc_kernels():
    mesh = plsc.VectorSubcoreMesh(core_axis_name="c", subcore_axis_name="s",
                                  num_cores=NC, num_subcores=NS)
    gather = pl.kernel(
        _sc_gather_body,
        out_type=(
            jax.ShapeDtypeStruct((EE, F), jnp.float32),
            jax.ShapeDtypeStruct((NW, NN), jnp.float32),
            jax.ShapeDtypeStruct((NW, NB * C), jnp.float32),
        ),
        mesh=mesh,
        compiler_params=pltpu.CompilerParams(use_tc_tiling_on_sc=False, needs_layout_passes=False),
        scratch_types=[
            pltpu.VMEM((CH,), jnp.int32),
            pltpu.VMEM((CH,), jnp.int32),
            pltpu.VMEM((CH,), jnp.int32),
            pltpu.VMEM((CH * C,), jnp.float32),
            pltpu.VMEM((CH, F), jnp.float32),
            pltpu.VMEM((NN,), jnp.int32),
            pltpu.VMEM((NN,), jnp.float32),
            pltpu.VMEM((NB * C,), jnp.float32),
            pltpu.SemaphoreType.DMA,
        ],
    )
    scatter = pl.kernel(
        _sc_scatter_body,
        out_type=jax.ShapeDtypeStruct((NC * NNP, F), jnp.float32),
        mesh=mesh,
        compiler_params=pltpu.CompilerParams(use_tc_tiling_on_sc=False, needs_layout_passes=False),
        scratch_types=[
            pltpu.VMEM((CH,), jnp.int32),
            pltpu.VMEM((CH, F), jnp.float32),
            pltpu.VMEM_SHARED((NNP, F), jnp.float32),
            pltpu.SemaphoreType.DMA,
        ],
    )
    return gather, scatter


def _relu(v):
    return jnp.maximum(v, 0.0)


def _dot(a, b):
    return jnp.dot(a, b, preferred_element_type=jnp.float32)


def _dg(a, b, ca, cb):
    return lax.dot_general(a, b, (((ca,), (cb,)), ((), ())),
                           preferred_element_type=jnp.float32)


def _encinit_body(x_ref, w0, b0, w1, b1, w2, b2, h_ref):
    h = _relu(_dot(x_ref[...], w0[...]) + b0[...])
    h = _relu(_dot(h, w1[...]) + b1[...])
    h_ref[...] = _dot(h, w2[...]) + b2[...]


def _msg_body(hs_ref, ea_ref, w2s_ref, bmat_ref, msg_ref):
    hs = hs_ref[...]
    acc = _dot(hs, bmat_ref[...])
    for c in range(C):
        acc = acc + _dot(hs * ea_ref[:, c][:, None], w2s_ref[c])
    msg_ref[...] = acc


def _enctail_body(x_ref, h_ref, aggp_ref, degp_ref, ecp_ref, batch_ref,
                  eps_ref, rootW, rootb, Wc, We4, Wa, Wm, Wx, Wml, Wxl,
                  b0r, W1r, b1r, Wfr, bfr,
                  mu_ref, lv_ref, lat_ref, ohT_s, cnt_s):
    x = x_ref[...]
    h = h_ref[...]
    ones_nw = jnp.full((NW, 1), 1.0, jnp.float32)
    deg = _dg(degp_ref[...], ones_nw, 0, 0)
    agg = (aggp_ref[0:NN, :] + aggp_ref[NNP:NNP + NN, :]) / jnp.maximum(deg, 1.0)
    new_x = _dot(h, rootW[...]) + rootb[...] + agg

    ec = jnp.sum(ecp_ref[...], axis=0) * (1.0 / NMAXC)

    batr = batch_ref[...]
    iota_bn = lax.broadcasted_iota(jnp.int32, (NB, NN), 0)
    ohT = jnp.where(iota_bn == batr, 1.0, 0.0)
    counts = jnp.sum(ohT, axis=1, keepdims=True)
    safe = jnp.maximum(counts, 1.0)
    addp = _dg(ohT, x, 1, 0)
    suml = _dg(ohT, new_x, 1, 0)
    meanp = addp / safe
    meanl = suml / safe

    ii = lax.broadcasted_iota(jnp.int32, (F, F), 0)
    jj = lax.broadcasted_iota(jnp.int32, (F, F), 1)
    eye = jnp.where(ii == jj, 1.0, 0.0)
    xT = _dg(eye, x, 1, 1)
    nxT = _dg(eye, new_x, 1, 1)

    ohT_s[...] = ohT
    cnt_s[...] = counts
    laneB = lax.broadcasted_iota(jnp.int32, (F, NB), 1)
    big = jnp.float32(-3.0e38)

    def mbody(b, accs):
        mp, ml = accs
        mask = ohT_s[pl.ds(b, 1), :] > 0.5
        cntb = cnt_s[pl.ds(b, 1), :]
        mx = jnp.max(jnp.where(mask, xT, big), axis=1, keepdims=True)
        mx = jnp.where(cntb > 0.5, mx, 0.0)
        mlx = jnp.max(jnp.where(mask, nxT, big), axis=1, keepdims=True)
        mlx = jnp.where(cntb > 0.5, mlx, 0.0)
        sel = laneB == b
        return (jnp.where(sel, mx, mp), jnp.where(sel, mlx, ml))

    zf = jnp.zeros((F, NB), jnp.float32)
    maxpT, maxlT = lax.fori_loop(0, NB, mbody, (zf, zf))

    h0 = (counts * (1.0 / NMAXC)) * Wc[...]
    h0 = h0 + _dot(ec, We4[...])
    h0 = h0 + _dot(addp * (1.0 / NMAXC), Wa[...])
    h0 = h0 + _dot(meanp, Wm[...])
    h0 = h0 + _dg(maxpT, Wx[...], 0, 0)
    h0 = h0 + _dot(meanl, Wml[...])
    h0 = h0 + _dg(maxlT, Wxl[...], 0, 0)
    h0 = _relu(h0 + b0r[...])
    h1 = _relu(_dot(h0, W1r[...]) + b1r[...])
    eo = _dot(h1, Wfr[...]) + bfr[...]
    mu = eo[:, :LATD]
    lv = eo[:, LATD:]
    mu_ref[...] = mu
    lv_ref[...] = lv
    lat_ref[...] = mu + eps_ref[...] * jnp.exp(0.5 * lv)


def _dec_body(lat_ref, pts_ref, Sj_ref,
              W0, b0, W0b, b0b, Wh, bh, Wfin, bfin,
              Wd0, bd0, Wd1, bd1,
              Wl0, bl0, Wl1, bl1, Wlf, blf,
              WeA, WeB, be0, We1, be1, Wef, befT,
              WaP, WaS, ba0, Wa1, ba1, Waf, baf,
              la_ref, ep_ref, et_ref):
    lat = lat_ref[...].reshape(GDEC, LATD)
    y = lat[:, COSD:]
    base = _dot(pts_ref[...], W0[...]) + b0[...] + b0b[...]
    yW = _dot(y, W0b[...])
    t = _relu((base[None, :, :] + yW[:, None, :]).reshape(GDEC * NMAXC, HD))
    t = _relu(_dot(t, Wh[...]) + bh[...])
    z = _relu(_dot(t, Wfin[...]) + bfin[...])
    z = z + _relu(_dot(z, Wd0[...]) + bd0[...])
    z = z + _relu(_dot(z, Wd1[...]) + bd1[...])
    t2 = _relu(_dot(z, Wl0[...]) + bl0[...])
    t2 = _relu(_dot(t2, Wl1[...]) + bl1[...])
    pos = _dot(t2, Wlf[...]) + blf[...]

    h1 = _dot(pos, WeA[...]) + be0[...]
    h2 = _dot(pos, WeB[...])
    M2 = NMAXC * NMAXC
    hh_parts = []
    for g in range(GDEC):
        h1g = h1[g * NMAXC:(g + 1) * NMAXC]
        h2g = h2[g * NMAXC:(g + 1) * NMAXC]
        hh_parts.append((h1g[:, None, :] + h2g[None, :, :]).reshape(M2, HE))
    hh = _relu(jnp.concatenate(hh_parts, axis=0))
    hh = _relu(_dot(hh, We1[...]) + be1[...])
    lgT = _dg(Wef[...], hh, 0, 1) + befT[...]
    l4 = lgT[:C, :]
    l5 = lgT[C:, :]
    m4 = jnp.max(l4, axis=0, keepdims=True)
    s4 = jnp.sum(jnp.exp(l4 - m4), axis=0, keepdims=True)
    etT = l4 - m4 - jnp.log(s4)
    m5 = jnp.maximum(m4, l5)
    s5 = jnp.sum(jnp.exp(l4 - m5), axis=0, keepdims=True) + jnp.exp(l5 - m5)
    epT = 1.0 - jnp.exp(l5 - m5) / s5

    sump_parts = [_dg(etT[:, g * M2:(g + 1) * M2], Sj_ref[...], 1, 0)
                  for g in range(GDEC)]
    sump2 = jnp.concatenate(sump_parts, axis=1) * (1.0 / 9.0)
    a0 = _relu(_dot(pos, WaP[...]) + _dg(sump2, WaS[...], 0, 0) + ba0[...])
    a1 = _relu(_dot(a0, Wa1[...]) + ba1[...])
    la = _dot(a1, Waf[...]) + baf[...]
    mla = jnp.max(la, axis=1, keepdims=True)
    la = la - mla - jnp.log(jnp.sum(jnp.exp(la - mla), axis=1, keepdims=True))

    for g in range(GDEC):
        la_ref[g] = la[g * NMAXC:(g + 1) * NMAXC]
        ep_ref[g] = epT[:, g * M2:(g + 1) * M2]
        et_ref[g] = etT[:, g * M2:(g + 1) * M2]


def _row(b):
    return b.reshape(1, -1)


def kernel(x, edge_index, edge_attr, batch, idx, params):
    f32 = jnp.float32
    src = edge_index[0].astype(jnp.int32)
    dst = edge_index[1].astype(jnp.int32)
    eaf = edge_attr.reshape(-1)
    batch = batch.astype(jnp.int32)
    zn = jnp.zeros((NN,), f32)
    zec = jnp.zeros((NB * C,), f32)
    zn16 = jnp.zeros((NNP, F), f32)
    eps = jax.random.normal(jax.random.key(1), (NB, LATD), dtype=f32)

    p = params
    ei = p["enc_init"]
    ef = p["enc_final"]
    W2s = p["nn_W"].reshape(C, F, F)
    Bmat = p["nn_b"].reshape(F, F)
    Wefin = ef["lin0"][0]

    h = pl.pallas_call(
        _encinit_body,
        out_shape=jax.ShapeDtypeStruct((NN, F), f32),
    )(x, ei["lin0"][0], _row(ei["lin0"][1]),
      ei["hidden"][0][0], _row(ei["hidden"][0][1]),
      ei["fin"][0], _row(ei["fin"][1]))

    sc_gather, sc_scatter = _sc_kernels()
    hsrc, degp, ecp = sc_gather(h, src, dst, eaf, batch, zn, zec)

    BE = 8000
    msg = pl.pallas_call(
        _msg_body,
        grid=(EE // BE,),
        in_specs=[
            pl.BlockSpec((BE, F), lambda e: (e, 0)),
            pl.BlockSpec((BE, C), lambda e: (e, 0)),
            pl.BlockSpec((C, F, F), lambda e: (0, 0, 0)),
            pl.BlockSpec((F, F), lambda e: (0, 0)),
        ],
        out_specs=pl.BlockSpec((BE, F), lambda e: (e, 0)),
        out_shape=jax.ShapeDtypeStruct((EE, F), f32),
    )(hsrc, edge_attr, W2s, Bmat)

    aggp = sc_scatter(msg, dst, zn16)

    mu, log_var, latent = pl.pallas_call(
        _enctail_body,
        out_shape=(
            jax.ShapeDtypeStruct((NB, LATD), f32),
            jax.ShapeDtypeStruct((NB, LATD), f32),
            jax.ShapeDtypeStruct((NB, LATD), f32),
        ),
        scratch_shapes=[
            pltpu.VMEM((NB, NN), f32),
            pltpu.VMEM((NB, 1), f32),
        ],
    )(x, h, aggp, degp, ecp.reshape(NW, NB, C), batch.reshape(1, NN), eps,
      p["root_W"], _row(p["root_b"]),
      Wefin[0:1], Wefin[1:1 + C], Wefin[5:21], Wefin[21:37], Wefin[37:53],
      Wefin[53:69], Wefin[69:85],
      _row(ef["lin0"][1]), ef["hidden"][0][0], _row(ef["hidden"][0][1]),
      ef["fin"][0], _row(ef["fin"][1]))

    di = p["dec_init"]
    ls = p["last_set"]
    em = p["edge_mlp"]
    am = p["atom_mlp"]
    Sj = (jnp.arange(NMAXC * NMAXC, dtype=jnp.int32)[:, None] // NMAXC
          == jnp.arange(NMAXC, dtype=jnp.int32)[None, :]).astype(f32)
    full = lambda *s: pl.BlockSpec(s, lambda g: tuple(0 for _ in s))

    la_out, ep_out, etT_out = pl.pallas_call(
        _dec_body,
        grid=(NB // GDEC,),
        in_specs=[
            pl.BlockSpec((GDEC, 1, LATD), lambda g: (g, 0, 0)),
            full(NMAXC, SETC_),
            full(NMAXC * NMAXC, NMAXC),
            full(SETC_, HD), full(1, HD),
            full(LATD - COSD, HD), full(1, HD),
            full(HD, HD), full(1, HD),
            full(HD, HD), full(1, HD),
            full(HD, HD), full(1, HD),
            full(HD, HD), full(1, HD),
            full(HD, HD), full(1, HD),
            full(HD, HD), full(1, HD),
            full(HD, SD), full(1, SD),
            full(SD, HE), full(SD, HE), full(1, HE),
            full(HE, HE), full(1, HE),
            full(HE, C + 1), full(C + 1, 1),
            full(SD, HD), full(C, HD), full(1, HD),
            full(HD, HD), full(1, HD),
            full(HD, F), full(1, F),
        ],
        out_specs=(
            pl.BlockSpec((GDEC, NMAXC, F), lambda g: (g, 0, 0)),
            pl.BlockSpec((GDEC, 1, NMAXC * NMAXC), lambda g: (g, 0, 0)),
            pl.BlockSpec((GDEC, C, NMAXC * NMAXC), lambda g: (g, 0, 0)),
        ),
        out_shape=(
            jax.ShapeDtypeStruct((NB, NMAXC, F), f32),
            jax.ShapeDtypeStruct((NB, 1, NMAXC * NMAXC), f32),
            jax.ShapeDtypeStruct((NB, C, NMAXC * NMAXC), f32),
        ),
    )(latent.reshape(NB, 1, LATD), p["points"], Sj,
      di["lin0"][0], _row(di["lin0"][1]),
      di["lin0b"][0], _row(di["lin0b"][1]),
      di["hidden"][0][0], _row(di["hidden"][0][1]),
      di["fin"][0], _row(di["fin"][1]),
      p["dec_lin0"][0], _row(p["dec_lin0"][1]),
      p["dec_lin1"][0], _row(p["dec_lin1"][1]),
      ls["lin0"][0], _row(ls["lin0"][1]),
      ls["hidden"][0][0], _row(ls["hidden"][0][1]),
      ls["fin"][0], _row(ls["fin"][1]),
      em["lin0"][0][:SD], em["lin0"][0][SD:], _row(em["lin0"][1]),
      em["hidden"][0][0], _row(em["hidden"][0][1]),
      em["fin"][0], em["fin"][1].reshape(C + 1, 1),
      am["lin0"][0][:SD], am["lin0"][0][SD:], _row(am["lin0"][1]),
      am["hidden"][0][0], _row(am["hidden"][0][1]),
      am["fin"][0], _row(am["fin"][1]))

    edge_probs = ep_out.reshape(NB, NMAXC, NMAXC)
    edge_types = jnp.transpose(etT_out, (0, 2, 1)).reshape(NB, NMAXC, NMAXC, C)
    return (la_out, edge_probs, edge_types, mu, log_var)


# speedup vs baseline: 2.5446x; 1.0075x over previous
import functools
import jax
import jax.numpy as jnp
from jax import lax
from jax.experimental import pallas as pl
from jax.experimental.pallas import tpu as pltpu
from jax.experimental.pallas import tpu_sc as plsc

NN = 10000
EE = 160000
NB = 100
NMAXC = 100
F = 16
C = 4
LATD = 64
COSD = 8
SETC_ = 32
HD = 64
HE = 32
SD = 3

NC = 2
NS = 16
NW = NC * NS
LANES = 16
CH = 128
NCH = EE // CH
TPW = (NCH + NW - 1) // NW
NNP = 10240
RPW = NNP // NS
GDEC = 4

def _sc_gather_body(h_hbm, src_hbm, dst_hbm, eaf_hbm, batch_hbm, zn_hbm,
                    zec_hbm, hsrc_hbm, degp_hbm, ecp_hbm,
                    sidx, didx, bsrcv, eav, rows, btab, dtab, ectab, sem):
    wid = lax.axis_index("s") * NC + lax.axis_index("c")
    pltpu.sync_copy(batch_hbm, btab)
    pltpu.sync_copy(zn_hbm, dtab)
    pltpu.sync_copy(zec_hbm, ectab)
    iota = lax.iota(jnp.int32, LANES)
    rep4 = iota // 4
    mod4 = iota - rep4 * 4
    ones = jnp.full((LANES,), 1.0, jnp.float32)

    def body(t, carry):
        ch = t * NW + wid

        @pl.when(ch < NCH)
        def _():
            e0 = ch * CH
            pltpu.sync_copy(src_hbm.at[pl.ds(e0, CH)], sidx)
            pltpu.sync_copy(dst_hbm.at[pl.ds(e0, CH)], didx)
            pltpu.sync_copy(eaf_hbm.at[pl.ds(e0 * C, CH * C)], eav)
            pltpu.async_copy(h_hbm.at[sidx], rows, sem).wait()
            pltpu.sync_copy(rows, hsrc_hbm.at[pl.ds(e0, CH)])
            for j in range(CH // LANES):
                d16 = didx[pl.ds(j * LANES, LANES)]
                plsc.addupdate_scatter(dtab, [d16], ones)
                s16 = sidx[pl.ds(j * LANES, LANES)]
                bsrcv[pl.ds(j * LANES, LANES)] = plsc.load_gather(btab, [s16])
            for g in range(CH // 4):
                ridx = g * 4 + rep4
                bg = plsc.load_gather(bsrcv, [ridx])
                vals = eav[pl.ds(g * LANES, LANES)]
                plsc.addupdate_scatter(ectab, [bg * C + mod4], vals)

        return carry

    lax.fori_loop(0, TPW, body, 0)
    pltpu.sync_copy(dtab, degp_hbm.at[wid])
    pltpu.sync_copy(ectab, ecp_hbm.at[wid])


def _sc_scatter_body(msg_hbm, dst_hbm, zn16_hbm, aggp_hbm,
                     didx, rowsv, shacc, sem):
    cid = lax.axis_index("c")
    sid = lax.axis_index("s")
    wid = sid * NC + cid
    rpw = RPW

    pltpu.sync_copy(zn16_hbm.at[pl.ds(sid * rpw, rpw)],
                    shacc.at[pl.ds(sid * rpw, rpw)])
    plsc.subcore_barrier()

    def body(t, carry):
        ch = t * NW + wid

        @pl.when(ch < NCH)
        def _():
            e0 = ch * CH
            pltpu.sync_copy(dst_hbm.at[pl.ds(e0, CH)], didx)
            pltpu.sync_copy(msg_hbm.at[pl.ds(e0, CH)], rowsv)
            pltpu.sync_copy(rowsv, shacc.at[didx], add=True)

        return carry

    lax.fori_loop(0, TPW, body, 0)
    plsc.subcore_barrier()
    pltpu.sync_copy(shacc.at[pl.ds(sid * rpw, rpw)],
                    aggp_hbm.at[pl.ds(cid * NNP + sid * rpw, rpw)])


@functools.cache
def _sc_kernels():
    mesh = plsc.VectorSubcoreMesh(core_axis_name="c", subcore_axis_name="s",
                                  num_cores=NC, num_subcores=NS)
    gather = pl.kernel(
        _sc_gather_body,
        out_type=(
            jax.ShapeDtypeStruct((EE, F), jnp.float32),
            jax.ShapeDtypeStruct((NW, NN), jnp.float32),
            jax.ShapeDtypeStruct((NW, NB * C), jnp.float32),
        ),
        mesh=mesh,
        compiler_params=pltpu.CompilerParams(use_tc_tiling_on_sc=False, needs_layout_passes=False),
        scratch_types=[
            pltpu.VMEM((CH,), jnp.int32),
            pltpu.VMEM((CH,), jnp.int32),
            pltpu.VMEM((CH,), jnp.int32),
            pltpu.VMEM((CH * C,), jnp.float32),
            pltpu.VMEM((CH, F), jnp.float32),
            pltpu.VMEM((NN,), jnp.int32),
            pltpu.VMEM((NN,), jnp.float32),
            pltpu.VMEM((NB * C,), jnp.float32),
            pltpu.SemaphoreType.DMA,
        ],
    )
    scatter = pl.kernel(
        _sc_scatter_body,
        out_type=jax.ShapeDtypeStruct((NC * NNP, F), jnp.float32),
        mesh=mesh,
        compiler_params=pltpu.CompilerParams(use_tc_tiling_on_sc=False, needs_layout_passes=False),
        scratch_types=[
            pltpu.VMEM((CH,), jnp.int32),
            pltpu.VMEM((CH, F), jnp.float32),
            pltpu.VMEM_SHARED((NNP, F), jnp.float32),
            pltpu.SemaphoreType.DMA,
        ],
    )
    return gather, scatter


def _relu(v):
    return jnp.maximum(v, 0.0)


def _dot(a, b):
    return jnp.dot(a, b, preferred_element_type=jnp.float32)


def _dg(a, b, ca, cb):
    return lax.dot_general(a, b, (((ca,), (cb,)), ((), ())),
                           preferred_element_type=jnp.float32)


def _encinit_body(x_ref, w0, b0, w1, b1, w2, b2, h_ref):
    h = _relu(_dot(x_ref[...], w0[...]) + b0[...])
    h = _relu(_dot(h, w1[...]) + b1[...])
    h_ref[...] = _dot(h, w2[...]) + b2[...]


def _msg_body(hs_ref, ea_ref, w2s_ref, bmat_ref, msg_ref):
    hs = hs_ref[...]
    acc = _dot(hs, bmat_ref[...])
    for c in range(C):
        acc = acc + _dot(hs * ea_ref[:, c][:, None], w2s_ref[c])
    msg_ref[...] = acc


def _enctail_body(x_ref, h_ref, aggp_ref, degp_ref, ecp_ref, batch_ref,
                  eps_ref, rootW, rootb, Wc, We4, Wa, Wm, Wx, Wml, Wxl,
                  b0r, W1r, b1r, Wfr, bfr,
                  mu_ref, lv_ref, lat_ref, ohT_s, cnt_s):
    x = x_ref[...]
    h = h_ref[...]
    ones_nw = jnp.full((NW, 1), 1.0, jnp.float32)
    deg = _dg(degp_ref[...], ones_nw, 0, 0)
    agg = (aggp_ref[0:NN, :] + aggp_ref[NNP:NNP + NN, :]) / jnp.maximum(deg, 1.0)
    new_x = _dot(h, rootW[...]) + rootb[...] + agg

    ec = jnp.sum(ecp_ref[...], axis=0) * (1.0 / NMAXC)

    batr = batch_ref[...]
    iota_bn = lax.broadcasted_iota(jnp.int32, (NB, NN), 0)
    ohT = jnp.where(iota_bn == batr, 1.0, 0.0)
    counts = jnp.sum(ohT, axis=1, keepdims=True)
    safe = jnp.maximum(counts, 1.0)
    addp = _dg(ohT, x, 1, 0)
    suml = _dg(ohT, new_x, 1, 0)
    meanp = addp / safe
    meanl = suml / safe

    ii = lax.broadcasted_iota(jnp.int32, (F, F), 0)
    jj = lax.broadcasted_iota(jnp.int32, (F, F), 1)
    eye = jnp.where(ii == jj, 1.0, 0.0)
    xT = _dg(eye, x, 1, 1)
    nxT = _dg(eye, new_x, 1, 1)

    ohT_s[...] = ohT
    cnt_s[...] = counts
    laneB = lax.broadcasted_iota(jnp.int32, (F, NB), 1)
    big = jnp.float32(-3.0e38)

    def mbody(b, accs):
        mp, ml = accs
        mask = ohT_s[pl.ds(b, 1), :] > 0.5
        cntb = cnt_s[pl.ds(b, 1), :]
        mx = jnp.max(jnp.where(mask, xT, big), axis=1, keepdims=True)
        mx = jnp.where(cntb > 0.5, mx, 0.0)
        mlx = jnp.max(jnp.where(mask, nxT, big), axis=1, keepdims=True)
        mlx = jnp.where(cntb > 0.5, mlx, 0.0)
        sel = laneB == b
        return (jnp.where(sel, mx, mp), jnp.where(sel, mlx, ml))

    zf = jnp.zeros((F, NB), jnp.float32)
    maxpT, maxlT = lax.fori_loop(0, NB, mbody, (zf, zf))

    h0 = (counts * (1.0 / NMAXC)) * Wc[...]
    h0 = h0 + _dot(ec, We4[...])
    h0 = h0 + _dot(addp * (1.0 / NMAXC), Wa[...])
    h0 = h0 + _dot(meanp, Wm[...])
    h0 = h0 + _dg(maxpT, Wx[...], 0, 0)
    h0 = h0 + _dot(meanl, Wml[...])
    h0 = h0 + _dg(maxlT, Wxl[...], 0, 0)
    h0 = _relu(h0 + b0r[...])
    h1 = _relu(_dot(h0, W1r[...]) + b1r[...])
    eo = _dot(h1, Wfr[...]) + bfr[...]
    mu = eo[:, :LATD]
    lv = eo[:, LATD:]
    mu_ref[...] = mu
    lv_ref[...] = lv
    lat_ref[...] = mu + eps_ref[...] * jnp.exp(0.5 * lv)


def _dec_body(lat_ref, pts_ref, Sj_ref,
              W0, b0, W0b, b0b, Wh, bh, Wfin, bfin,
              Wd0, bd0, Wd1, bd1,
              Wl0, bl0, Wl1, bl1, Wlf, blf,
              WeA, WeB, be0, We1, be1, Wef, befT,
              WaP, WaS, ba0, Wa1, ba1, Waf, baf,
              la_ref, ep_ref, et_ref):
    lat = lat_ref[...].reshape(GDEC, LATD)
    y = lat[:, COSD:]
    base = _dot(pts_ref[...], W0[...]) + b0[...] + b0b[...]
    yW = _dot(y, W0b[...])
    t = _relu((base[None, :, :] + yW[:, None, :]).reshape(GDEC * NMAXC, HD))
    t = _relu(_dot(t, Wh[...]) + bh[...])
    z = _relu(_dot(t, Wfin[...]) + bfin[...])
    z = z + _relu(_dot(z, Wd0[...]) + bd0[...])
    z = z + _relu(_dot(z, Wd1[...]) + bd1[...])
    t2 = _relu(_dot(z, Wl0[...]) + bl0[...])
    t2 = _relu(_dot(t2, Wl1[...]) + bl1[...])
    pos = _dot(t2, Wlf[...]) + blf[...]

    h1 = _dot(pos, WeA[...]) + be0[...]
    h2 = _dot(pos, WeB[...])
    M2 = NMAXC * NMAXC
    hh_parts = []
    for g in range(GDEC):
        h1g = h1[g * NMAXC:(g + 1) * NMAXC]
        h2g = h2[g * NMAXC:(g + 1) * NMAXC]
        hh_parts.append((h1g[:, None, :] + h2g[None, :, :]).reshape(M2, HE))
    hh = _relu(jnp.concatenate(hh_parts, axis=0))
    hh = _relu(_dot(hh, We1[...]) + be1[...])
    lgT = _dg(Wef[...], hh, 0, 1) + befT[...]
    l4 = lgT[:C, :]
    l5 = lgT[C:, :]
    m4 = jnp.max(l4, axis=0, keepdims=True)
    s4 = jnp.sum(jnp.exp(l4 - m4), axis=0, keepdims=True)
    etT = l4 - m4 - jnp.log(s4)
    m5 = jnp.maximum(m4, l5)
    s5 = jnp.sum(jnp.exp(l4 - m5), axis=0, keepdims=True) + jnp.exp(l5 - m5)
    epT = 1.0 - jnp.exp(l5 - m5) / s5

    sump_parts = [_dg(etT[:, g * M2:(g + 1) * M2], Sj_ref[...], 1, 0)
                  for g in range(GDEC)]
    sump2 = jnp.concatenate(sump_parts, axis=1) * (1.0 / 9.0)
    a0 = _relu(_dot(pos, WaP[...]) + _dg(sump2, WaS[...], 0, 0) + ba0[...])
    a1 = _relu(_dot(a0, Wa1[...]) + ba1[...])
    la = _dot(a1, Waf[...]) + baf[...]
    mla = jnp.max(la, axis=1, keepdims=True)
    la = la - mla - jnp.log(jnp.sum(jnp.exp(la - mla), axis=1, keepdims=True))

    for g in range(GDEC):
        la_ref[g] = la[g * NMAXC:(g + 1) * NMAXC]
        ep_ref[g] = epT[:, g * M2:(g + 1) * M2]
        et_ref[g] = etT[:, g * M2:(g + 1) * M2]


def _row(b):
    return b.reshape(1, -1)


def kernel(x, edge_index, edge_attr, batch, idx, params):
    f32 = jnp.float32
    src = edge_index[0].astype(jnp.int32)
    dst = edge_index[1].astype(jnp.int32)
    eaf = edge_attr.reshape(-1)
    batch = batch.astype(jnp.int32)
    zn = jnp.zeros((NN,), f32)
    zec = jnp.zeros((NB * C,), f32)
    zn16 = jnp.zeros((NNP, F), f32)
    eps = jax.random.normal(jax.random.key(1), (NB, LATD), dtype=f32)

    p = params
    ei = p["enc_init"]
    ef = p["enc_final"]
    W2s = p["nn_W"].reshape(C, F, F)
    Bmat = p["nn_b"].reshape(F, F)
    Wefin = ef["lin0"][0]

    h = pl.pallas_call(
        _encinit_body,
        out_shape=jax.ShapeDtypeStruct((NN, F), f32),
    )(x, ei["lin0"][0], _row(ei["lin0"][1]),
      ei["hidden"][0][0], _row(ei["hidden"][0][1]),
      ei["fin"][0], _row(ei["fin"][1]))

    sc_gather, sc_scatter = _sc_kernels()
    hsrc, degp, ecp = sc_gather(h, src, dst, eaf, batch, zn, zec)

    BE = 16000
    msg = pl.pallas_call(
        _msg_body,
        grid=(EE // BE,),
        in_specs=[
            pl.BlockSpec((BE, F), lambda e: (e, 0)),
            pl.BlockSpec((BE, C), lambda e: (e, 0)),
            pl.BlockSpec((C, F, F), lambda e: (0, 0, 0)),
            pl.BlockSpec((F, F), lambda e: (0, 0)),
        ],
        out_specs=pl.BlockSpec((BE, F), lambda e: (e, 0)),
        out_shape=jax.ShapeDtypeStruct((EE, F), f32),
    )(hsrc, edge_attr, W2s, Bmat)

    aggp = sc_scatter(msg, dst, zn16)

    mu, log_var, latent = pl.pallas_call(
        _enctail_body,
        out_shape=(
            jax.ShapeDtypeStruct((NB, LATD), f32),
            jax.ShapeDtypeStruct((NB, LATD), f32),
            jax.ShapeDtypeStruct((NB, LATD), f32),
        ),
        scratch_shapes=[
            pltpu.VMEM((NB, NN), f32),
            pltpu.VMEM((NB, 1), f32),
        ],
    )(x, h, aggp, degp, ecp.reshape(NW, NB, C), batch.reshape(1, NN), eps,
      p["root_W"], _row(p["root_b"]),
      Wefin[0:1], Wefin[1:1 + C], Wefin[5:21], Wefin[21:37], Wefin[37:53],
      Wefin[53:69], Wefin[69:85],
      _row(ef["lin0"][1]), ef["hidden"][0][0], _row(ef["hidden"][0][1]),
      ef["fin"][0], _row(ef["fin"][1]))

    di = p["dec_init"]
    ls = p["last_set"]
    em = p["edge_mlp"]
    am = p["atom_mlp"]
    Sj = (jnp.arange(NMAXC * NMAXC, dtype=jnp.int32)[:, None] // NMAXC
          == jnp.arange(NMAXC, dtype=jnp.int32)[None, :]).astype(f32)
    full = lambda *s: pl.BlockSpec(s, lambda g: tuple(0 for _ in s))

    la_out, ep_out, etT_out = pl.pallas_call(
        _dec_body,
        grid=(NB // GDEC,),
        in_specs=[
            pl.BlockSpec((GDEC, 1, LATD), lambda g: (g, 0, 0)),
            full(NMAXC, SETC_),
            full(NMAXC * NMAXC, NMAXC),
            full(SETC_, HD), full(1, HD),
            full(LATD - COSD, HD), full(1, HD),
            full(HD, HD), full(1, HD),
            full(HD, HD), full(1, HD),
            full(HD, HD), full(1, HD),
            full(HD, HD), full(1, HD),
            full(HD, HD), full(1, HD),
            full(HD, HD), full(1, HD),
            full(HD, SD), full(1, SD),
            full(SD, HE), full(SD, HE), full(1, HE),
            full(HE, HE), full(1, HE),
            full(HE, C + 1), full(C + 1, 1),
            full(SD, HD), full(C, HD), full(1, HD),
            full(HD, HD), full(1, HD),
            full(HD, F), full(1, F),
        ],
        out_specs=(
            pl.BlockSpec((GDEC, NMAXC, F), lambda g: (g, 0, 0)),
            pl.BlockSpec((GDEC, 1, NMAXC * NMAXC), lambda g: (g, 0, 0)),
            pl.BlockSpec((GDEC, C, NMAXC * NMAXC), lambda g: (g, 0, 0)),
        ),
        out_shape=(
            jax.ShapeDtypeStruct((NB, NMAXC, F), f32),
            jax.ShapeDtypeStruct((NB, 1, NMAXC * NMAXC), f32),
            jax.ShapeDtypeStruct((NB, C, NMAXC * NMAXC), f32),
        ),
    )(latent.reshape(NB, 1, LATD), p["points"], Sj,
      di["lin0"][0], _row(di["lin0"][1]),
      di["lin0b"][0], _row(di["lin0b"][1]),
      di["hidden"][0][0], _row(di["hidden"][0][1]),
      di["fin"][0], _row(di["fin"][1]),
      p["dec_lin0"][0], _row(p["dec_lin0"][1]),
      p["dec_lin1"][0], _row(p["dec_lin1"][1]),
      ls["lin0"][0], _row(ls["lin0"][1]),
      ls["hidden"][0][0], _row(ls["hidden"][0][1]),
      ls["fin"][0], _row(ls["fin"][1]),
      em["lin0"][0][:SD], em["lin0"][0][SD:], _row(em["lin0"][1]),
      em["hidden"][0][0], _row(em["hidden"][0][1]),
      em["fin"][0], em["fin"][1].reshape(C + 1, 1),
      am["lin0"][0][:SD], am["lin0"][0][SD:], _row(am["lin0"][1]),
      am["hidden"][0][0], _row(am["hidden"][0][1]),
      am["fin"][0], _row(am["fin"][1]))

    edge_probs = ep_out.reshape(NB, NMAXC, NMAXC)
    edge_types = jnp.transpose(etT_out, (0, 2, 1)).reshape(NB, NMAXC, NMAXC, C)
    return (la_out, edge_probs, edge_types, mu, log_var)
